# baseline (speedup 1.0000x reference)
import functools

import numpy as np
import jax
import jax.numpy as jnp
from jax import lax
from jax.experimental import pallas as pl
from jax.experimental.pallas import tpu as pltpu
from jax.experimental.pallas import tpu_sc as plsc

VOCAB = 30522
DIM = 768
SEQ = 64
BATCH = 4096
LANES = 16

NC = 2
NS = 16
NW = NC * NS

TOTAL = BATCH * SEQ
ROWS_PER_W = TOTAL // NW
C = 32
NCHUNKS = ROWS_PER_W // C
VPR = DIM // LANES


def _pos_table():
    pos = np.arange(SEQ, dtype=np.float32)[:, None]
    i = np.arange(DIM // 2, dtype=np.float32)[None, :]
    angle = pos / np.power(10000.0, 2.0 * i / DIM)
    return np.concatenate([np.sin(angle), np.cos(angle)], axis=-1).astype(np.float32)


_MESH = plsc.VectorSubcoreMesh(core_axis_name="c", subcore_axis_name="s")


@functools.partial(
    pl.kernel,
    out_type=jax.ShapeDtypeStruct((TOTAL, DIM), jnp.float32),
    mesh=_MESH,
    scratch_types=[
        pltpu.VMEM((NCHUNKS, C), jnp.int32),
        pltpu.VMEM((C, DIM), jnp.float32),
        pltpu.VMEM((SEQ, DIM), jnp.float32),
        pltpu.SemaphoreType.DMA,
    ],
)
def _embed(x_hbm, wte_hbm, pos_hbm, out_hbm, idx_v, rows_v, pos_v, sem):
    wid = lax.axis_index("s") * NC + lax.axis_index("c")
    base = wid * ROWS_PER_W
    pltpu.sync_copy(pos_hbm, pos_v)
    pltpu.sync_copy(x_hbm.at[wid], idx_v)

    def do_chunk(g, carry):
        pltpu.async_copy(wte_hbm.at[idx_v.at[g]], rows_v, sem).wait()
        pbase = (g % (SEQ // C)) * C

        def add_row(r, c2):
            for j in range(VPR):
                sl = pl.ds(j * LANES, LANES)
                rows_v[r, sl] = rows_v[r, sl] + pos_v[pbase + r, sl]
            return c2

        lax.fori_loop(0, C, add_row, 0)
        pltpu.sync_copy(rows_v, out_hbm.at[pl.ds(base + g * C, C)])
        return carry

    lax.fori_loop(0, NCHUNKS, do_chunk, 0)


def kernel(x, wte):
    pos = jnp.asarray(_pos_table())
    idx = x.astype(jnp.int32).reshape(NW, NCHUNKS, C)
    out = _embed(idx, wte, pos)
    return out.reshape(BATCH, SEQ, DIM)


# speedup vs baseline: 4.3588x; 4.3588x over previous
import functools

import numpy as np
import jax
import jax.numpy as jnp
from jax import lax
from jax.experimental import pallas as pl
from jax.experimental.pallas import tpu as pltpu
from jax.experimental.pallas import tpu_sc as plsc

VOCAB = 30522
DIM = 768
SEQ = 64
BATCH = 4096
LANES = 16

NC = 2
NS = 16
NW = NC * NS

POS_PER_W = SEQ // NW
C = 32
NCH = BATCH // C
NBUF = 4
VPR = DIM // LANES


def _pos_table():
    pos = np.arange(SEQ, dtype=np.float32)[:, None]
    i = np.arange(DIM // 2, dtype=np.float32)[None, :]
    angle = pos / np.power(10000.0, 2.0 * i / DIM)
    return np.concatenate([np.sin(angle), np.cos(angle)], axis=-1).astype(np.float32)


_MESH = plsc.VectorSubcoreMesh(core_axis_name="c", subcore_axis_name="s")


@functools.partial(
    pl.kernel,
    out_type=jax.ShapeDtypeStruct((BATCH, SEQ, DIM), jnp.float32),
    mesh=_MESH,
    scratch_types=[
        pltpu.VMEM((NCH, C), jnp.int32),
        pltpu.VMEM((DIM,), jnp.float32),
        pltpu.VMEM((NBUF, C, DIM), jnp.float32),
        pltpu.SemaphoreType.DMA,
        pltpu.SemaphoreType.DMA,
        pltpu.SemaphoreType.DMA,
        pltpu.SemaphoreType.DMA,
        pltpu.SemaphoreType.DMA,
        pltpu.SemaphoreType.DMA,
        pltpu.SemaphoreType.DMA,
        pltpu.SemaphoreType.DMA,
    ],
)
def _embed(xt_hbm, wte_hbm, pos_hbm, out_hbm, idx_v, prow_v, ring_v,
           g0, g1, g2, g3, o0, o1, o2, o3):
    wid = lax.axis_index("s") * NC + lax.axis_index("c")
    s0 = wid * POS_PER_W
    gsems = (g0, g1, g2, g3)
    osems = (o0, o1, o2, o3)

    for bp in range(POS_PER_W):
        s = s0 + bp
        pltpu.sync_copy(xt_hbm.at[s], idx_v)
        pltpu.sync_copy(pos_hbm.at[s], prow_v)
        pvals = [prow_v[pl.ds(j * LANES, LANES)] for j in range(VPR)]

        def _gather(g, buf):
            return pltpu.make_async_copy(
                wte_hbm.at[idx_v.at[g]], ring_v.at[buf], gsems[buf])

        def _ocopy(g, buf):
            return pltpu.make_async_copy(
                ring_v.at[buf], out_hbm.at[pl.ds(g * C, C), s], osems[buf])

        _gather(0, 0).start()
        _gather(1, 1).start()

        def outer(i, carry):
            gbase = i * NBUF
            for b in range(NBUF):
                g = gbase + b
                _gather(g, b).wait()

                @pl.when(g >= 2)
                def _():
                    _ocopy(g - 2, (b + 2) % NBUF).wait()

                def add_row(r, c2):
                    for j in range(VPR):
                        sl = pl.ds(j * LANES, LANES)
                        ring_v[b, r, sl] = ring_v[b, r, sl] + pvals[j]
                    return c2

                lax.fori_loop(0, C, add_row, 0)

                @pl.when(g + 2 < NCH)
                def _():
                    _gather(g + 2, (b + 2) % NBUF).start()

                _ocopy(g, b).start()
            return carry

        lax.fori_loop(0, NCH // NBUF, outer, 0)

        _ocopy(NCH - 2, (NCH - 2) % NBUF).wait()
        _ocopy(NCH - 1, (NCH - 1) % NBUF).wait()


def kernel(x, wte):
    pos = jnp.asarray(_pos_table())
    xt = x.astype(jnp.int32).T.reshape(SEQ, NCH, C)
    return _embed(xt, wte, pos)
